# baseline (speedup 1.0000x reference)
import functools

import jax
import jax.numpy as jnp
from jax import lax
from jax.experimental import pallas as pl
from jax.experimental.pallas import tpu as pltpu
from jax.experimental.pallas import tpu_sc as plsc

TOKENS = 32768
D_IN = 768
D_HID = 768
E = 64
BT = 4096
NW = 32
RPW = D_HID // NW
CHUNK = RPW * D_IN


def _route_combine_sc(x_hbm, wgr_hbm, bgr_hbm, wflat_hbm, b_hbm,
                      wc_out, bc_out,
                      xv, bgv, w0buf, w1buf, bb0, bb1,
                      sem0, sem1):
    c = lax.axis_index("c")
    s = lax.axis_index("s")
    lanes = lax.broadcasted_iota(jnp.int32, (16,), 0)

    SLAB = D_IN * 16
    pltpu.sync_copy(x_hbm.at[0], xv)
    wgts = [w0buf, w1buf]
    sems = [sem0, sem1]
    cp = pltpu.async_copy(wgr_hbm.at[0], wgts[0].at[pl.ds(0, SLAB)], sems[0])
    ls = []
    for t in range(4):
        pltpu.sync_copy(bgr_hbm.at[t], bgv)
        cp.wait()
        if t < 3:
            cp = pltpu.async_copy(
                wgr_hbm.at[t + 1], wgts[(t + 1) % 2].at[pl.ds(0, SLAB)],
                sems[(t + 1) % 2])
        wgt = wgts[t % 2]

        def fma(j, acc, wgt=wgt):
            xc = xv[pl.ds(j * 16, 16)]
            for l in range(16):
                acc = acc + wgt[pl.ds((j * 16 + l) * 16, 16)] * xc[l]
            return acc

        ls.append(lax.fori_loop(0, D_IN // 16, fma, bgv[...]))

    ids = [lanes + 16 * t for t in range(4)]
    v, ix = ls[0], ids[0]
    for t in range(1, 4):
        hit = ls[t] > v
        ix = jnp.where(hit, ids[t], ix)
        v = jnp.where(hit, ls[t], v)
    m1 = jnp.max(v)
    i0 = jnp.min(jnp.where(v == m1, ix, E))
    ls2 = [jnp.where(ids[t] == i0, -1e30, ls[t]) for t in range(4)]
    v2, ix2 = ls2[0], ids[0]
    for t in range(1, 4):
        hit = ls2[t] > v2
        ix2 = jnp.where(hit, ids[t], ix2)
        v2 = jnp.where(hit, ls2[t], v2)
    m2 = jnp.max(v2)
    i1 = jnp.min(jnp.where(v2 == m2, ix2, E))
    esum = (jnp.exp(ls[0] - m1) + jnp.exp(ls[1] - m1)
            + jnp.exp(ls[2] - m1) + jnp.exp(ls[3] - m1))
    denomv = jnp.full((16,), jnp.sum(esum), jnp.float32)
    s0v = 1.0 / denomv
    s1v = jnp.exp(jnp.full((16,), m2 - m1, jnp.float32)) / denomv
    s0 = jnp.max(s0v)
    s1 = jnp.max(s1v)

    wid = c * 16 + s
    cp0 = pltpu.async_copy(wflat_hbm.at[i0, wid], w0buf, sem0)
    cp1 = pltpu.async_copy(wflat_hbm.at[i1, wid], w1buf, sem1)

    @pl.when((s == 0) & (c == 0))
    def _bias_fetch():
        pltpu.sync_copy(b_hbm.at[i0], bb0)
        pltpu.sync_copy(b_hbm.at[i1], bb1)

    cp0.wait()
    cp1.wait()

    def comb(j, _):
        for u in range(16):
            sl = pl.ds(j * 256 + u * 16, 16)
            w0buf[sl] = s0 * w0buf[sl] + s1 * w1buf[sl]
        return 0

    lax.fori_loop(0, CHUNK // 256, comb, 0)
    pltpu.sync_copy(w0buf, wc_out.at[wid])

    @pl.when((s == 0) & (c == 0))
    def _bias():
        def combb(j, _):
            sl = pl.ds(j * 16, 16)
            bb0[sl] = s0 * bb0[sl] + s1 * bb1[sl]
            return 0

        lax.fori_loop(0, D_HID // 16, combb, 0)
        pltpu.sync_copy(bb0, bc_out)


def _matmul_tc(x_ref, wc_ref, bc_ref, out_ref):
    out_ref[...] = jax.lax.dot_general(
        x_ref[...], wc_ref[...], (((1,), (1,)), ((), ())),
        preferred_element_type=jnp.float32) + bc_ref[...]


def kernel(x, W_experts, b_experts, Wg, bg):
    n_tokens = x.shape[0]
    wgr = Wg.T.reshape(D_IN, 4, 16).transpose(1, 0, 2).reshape(4, D_IN * 16)
    bgr = bg.reshape(4, 16)
    w_flat = W_experts.reshape(E, NW, CHUNK)

    sc_fn = pl.kernel(
        _route_combine_sc,
        out_type=(
            jax.ShapeDtypeStruct((NW, CHUNK), jnp.float32),
            jax.ShapeDtypeStruct((D_HID,), jnp.float32),
        ),
        mesh=plsc.VectorSubcoreMesh(core_axis_name="c", subcore_axis_name="s"),
        compiler_params=pltpu.CompilerParams(needs_layout_passes=False),
        scratch_types=[
            pltpu.VMEM((D_IN,), jnp.float32),
            pltpu.VMEM((16,), jnp.float32),
            pltpu.VMEM((CHUNK,), jnp.float32),
            pltpu.VMEM((CHUNK,), jnp.float32),
            pltpu.VMEM((D_HID,), jnp.float32),
            pltpu.VMEM((D_HID,), jnp.float32),
            pltpu.SemaphoreType.DMA,
            pltpu.SemaphoreType.DMA,
        ],
    )
    wc_flat, bc = sc_fn(x, wgr, bgr, w_flat, b_experts)
    wc = wc_flat.reshape(D_HID, D_IN)

    return pl.pallas_call(
        _matmul_tc,
        grid=(n_tokens // BT,),
        in_specs=[
            pl.BlockSpec((BT, D_IN), lambda i: (i, 0)),
            pl.BlockSpec((D_HID, D_IN), lambda i: (0, 0)),
            pl.BlockSpec((1, D_HID), lambda i: (0, 0)),
        ],
        out_specs=pl.BlockSpec((BT, D_HID), lambda i: (i, 0)),
        out_shape=jax.ShapeDtypeStruct((n_tokens, D_HID), jnp.float32),
    )(x, wc, bc.reshape(1, D_HID))


# speedup vs baseline: 2.3469x; 2.3469x over previous
import functools

import jax
import jax.numpy as jnp
from jax import lax
from jax.experimental import pallas as pl
from jax.experimental.pallas import tpu as pltpu
from jax.experimental.pallas import tpu_sc as plsc

TOKENS = 32768
D_IN = 768
D_HID = 768
E = 64
BT = 4096
NW = 32
RPW = D_HID // NW


def _route_combine_sc(x_hbm, wgr_hbm, bgr_hbm, wrows_hbm, b_hbm,
                      wc_out, bc_out,
                      xv, bgv, wbuf, idxv, oidx, bb0, bb1,
                      sem0, sem1):
    c = lax.axis_index("c")
    s = lax.axis_index("s")
    lanes = lax.broadcasted_iota(jnp.int32, (16,), 0)

    pltpu.sync_copy(x_hbm.at[0], xv)
    cp = pltpu.async_copy(wgr_hbm.at[0], wbuf.at[pl.ds(0, 16)], sem0)
    sems = [sem0, sem1]
    ls = []
    for t in range(4):
        roff = (t % 2) * 16
        pltpu.sync_copy(bgr_hbm.at[t], bgv)
        cp.wait()
        if t < 3:
            cp = pltpu.async_copy(
                wgr_hbm.at[t + 1], wbuf.at[pl.ds(((t + 1) % 2) * 16, 16)],
                sems[(t + 1) % 2])

        def fma(r, acc, roff=roff):
            xa = xv[pl.ds(r * 48, 16)]
            xb = xv[pl.ds(r * 48 + 16, 16)]
            xc = xv[pl.ds(r * 48 + 32, 16)]
            for dd in range(48):
                xs = (xa, xb, xc)[dd // 16][dd % 16]
                acc = acc + wbuf[roff + r, pl.ds(dd * 16, 16)] * xs
            return acc

        ls.append(lax.fori_loop(0, 16, fma, bgv[...]))

    ids = [lanes + 16 * t for t in range(4)]
    v, ix = ls[0], ids[0]
    for t in range(1, 4):
        hit = ls[t] > v
        ix = jnp.where(hit, ids[t], ix)
        v = jnp.where(hit, ls[t], v)
    m1 = jnp.max(v)
    i0 = jnp.min(jnp.where(v == m1, ix, E))
    ls2 = [jnp.where(ids[t] == i0, -1e30, ls[t]) for t in range(4)]
    v2, ix2 = ls2[0], ids[0]
    for t in range(1, 4):
        hit = ls2[t] > v2
        ix2 = jnp.where(hit, ids[t], ix2)
        v2 = jnp.where(hit, ls2[t], v2)
    m2 = jnp.max(v2)
    i1 = jnp.min(jnp.where(v2 == m2, ix2, E))
    esum = (jnp.exp(ls[0] - m1) + jnp.exp(ls[1] - m1)
            + jnp.exp(ls[2] - m1) + jnp.exp(ls[3] - m1))
    denomv = jnp.full((16,), jnp.sum(esum), jnp.float32)
    s0v = 1.0 / denomv
    s1v = jnp.exp(jnp.full((16,), m2 - m1, jnp.float32)) / denomv
    s0 = jnp.max(s0v)
    s1 = jnp.max(s1v)

    wid = c * 16 + s
    base = wid * RPW
    idxv[pl.ds(0, 16)] = i0 * D_HID + base + lanes
    idxv[pl.ds(16, 16)] = jnp.where(
        lanes < 8, i0 * D_HID + base + 16 + lanes,
        i1 * D_HID + base + (lanes - 8))
    idxv[pl.ds(32, 16)] = i1 * D_HID + base + 8 + lanes
    oidx[pl.ds(0, 16)] = base + lanes
    oidx[pl.ds(8, 16)] = base + 8 + lanes
    cpg = pltpu.async_copy(wrows_hbm.at[idxv], wbuf, sem0)

    @pl.when((s == 0) & (c == 0))
    def _bias_fetch():
        pltpu.sync_copy(b_hbm.at[i0], bb0)
        pltpu.sync_copy(b_hbm.at[i1], bb1)

    cpg.wait()

    def comb(r, _):
        for u in range(48):
            sl = pl.ds(u * 16, 16)
            wbuf[r, sl] = s0 * wbuf[r, sl] + s1 * wbuf[r + RPW, sl]
        return 0

    lax.fori_loop(0, RPW, comb, 0)
    pltpu.async_copy(wbuf.at[pl.ds(0, RPW)], wc_out.at[oidx], sem1).wait()

    @pl.when((s == 0) & (c == 0))
    def _bias():
        def combb(j, _):
            sl = pl.ds(j * 16, 16)
            bb0[sl] = s0 * bb0[sl] + s1 * bb1[sl]
            return 0

        lax.fori_loop(0, D_HID // 16, combb, 0)
        pltpu.sync_copy(bb0, bc_out)


def _matmul_tc(x_ref, wc_ref, bc_ref, out_ref):
    out_ref[...] = jax.lax.dot_general(
        x_ref[...], wc_ref[...], (((1,), (1,)), ((), ())),
        preferred_element_type=jnp.float32) + bc_ref[...]


def kernel(x, W_experts, b_experts, Wg, bg):
    n_tokens = x.shape[0]
    wgr = (Wg.T.reshape(16, 48, 4, 16).transpose(2, 0, 1, 3)
           .reshape(4, 16, D_IN))
    bgr = bg.reshape(4, 16)
    w_rows = W_experts.reshape(E * D_HID, D_IN)

    sc_fn = pl.kernel(
        _route_combine_sc,
        out_type=(
            jax.ShapeDtypeStruct((D_HID, D_IN), jnp.float32),
            jax.ShapeDtypeStruct((D_HID,), jnp.float32),
        ),
        mesh=plsc.VectorSubcoreMesh(core_axis_name="c", subcore_axis_name="s"),
        compiler_params=pltpu.CompilerParams(needs_layout_passes=False),
        scratch_types=[
            pltpu.VMEM((D_IN,), jnp.float32),
            pltpu.VMEM((16,), jnp.float32),
            pltpu.VMEM((2 * RPW, D_IN), jnp.float32),
            pltpu.VMEM((48,), jnp.int32),
            pltpu.VMEM((RPW,), jnp.int32),
            pltpu.VMEM((D_HID,), jnp.float32),
            pltpu.VMEM((D_HID,), jnp.float32),
            pltpu.SemaphoreType.DMA,
            pltpu.SemaphoreType.DMA,
        ],
    )
    wc, bc = sc_fn(x, wgr, bgr, w_rows, b_experts)

    return pl.pallas_call(
        _matmul_tc,
        grid=(n_tokens // BT,),
        in_specs=[
            pl.BlockSpec((BT, D_IN), lambda i: (i, 0)),
            pl.BlockSpec((D_HID, D_IN), lambda i: (0, 0)),
            pl.BlockSpec((1, D_HID), lambda i: (0, 0)),
        ],
        out_specs=pl.BlockSpec((BT, D_HID), lambda i: (i, 0)),
        out_shape=jax.ShapeDtypeStruct((n_tokens, D_HID), jnp.float32),
    )(x, wc, bc.reshape(1, D_HID))


# speedup vs baseline: 2.3804x; 1.0143x over previous
import functools

import jax
import jax.numpy as jnp
from jax import lax
from jax.experimental import pallas as pl
from jax.experimental.pallas import tpu as pltpu
from jax.experimental.pallas import tpu_sc as plsc

TOKENS = 32768
D_IN = 768
D_HID = 768
E = 64
BT = 4096
NW = 32
RPW = D_HID // NW


def _route_combine_sc(x_hbm, wgr_hbm, bgr_hbm, wrows_hbm, b_hbm,
                      wc_out, bc_out,
                      xv, bgv, wbuf, idxv, oidx, bb0, bb1,
                      sem0, sem1):
    c = lax.axis_index("c")
    s = lax.axis_index("s")
    lanes = lax.broadcasted_iota(jnp.int32, (16,), 0)

    pltpu.sync_copy(x_hbm.at[0], xv)
    cp = pltpu.async_copy(wgr_hbm.at[0], wbuf.at[pl.ds(0, 16)], sem0)
    sems = [sem0, sem1]
    ls = []
    for t in range(4):
        roff = (t % 2) * 16
        pltpu.sync_copy(bgr_hbm.at[t], bgv)
        cp.wait()
        if t < 3:
            cp = pltpu.async_copy(
                wgr_hbm.at[t + 1], wbuf.at[pl.ds(((t + 1) % 2) * 16, 16)],
                sems[(t + 1) % 2])

        def fma(r, accs, roff=roff):
            a0, a1, a2 = accs
            xa = xv[pl.ds(r * 48, 16)]
            xb = xv[pl.ds(r * 48 + 16, 16)]
            xc = xv[pl.ds(r * 48 + 32, 16)]
            for dd in range(16):
                a0 = a0 + wbuf[roff + r, pl.ds(dd * 16, 16)] * xa[dd]
                a1 = a1 + wbuf[roff + r, pl.ds((16 + dd) * 16, 16)] * xb[dd]
                a2 = a2 + wbuf[roff + r, pl.ds((32 + dd) * 16, 16)] * xc[dd]
            return a0, a1, a2

        z = jnp.zeros((16,), jnp.float32)
        a0, a1, a2 = lax.fori_loop(0, 16, fma, (bgv[...], z, z))
        ls.append(a0 + a1 + a2)

    ids = [lanes + 16 * t for t in range(4)]
    v, ix = ls[0], ids[0]
    for t in range(1, 4):
        hit = ls[t] > v
        ix = jnp.where(hit, ids[t], ix)
        v = jnp.where(hit, ls[t], v)
    m1 = jnp.max(v)
    i0 = jnp.min(jnp.where(v == m1, ix, E))
    ls2 = [jnp.where(ids[t] == i0, -1e30, ls[t]) for t in range(4)]
    v2, ix2 = ls2[0], ids[0]
    for t in range(1, 4):
        hit = ls2[t] > v2
        ix2 = jnp.where(hit, ids[t], ix2)
        v2 = jnp.where(hit, ls2[t], v2)
    m2 = jnp.max(v2)
    i1 = jnp.min(jnp.where(v2 == m2, ix2, E))
    esum = (jnp.exp(ls[0] - m1) + jnp.exp(ls[1] - m1)
            + jnp.exp(ls[2] - m1) + jnp.exp(ls[3] - m1))
    denomv = jnp.full((16,), jnp.sum(esum), jnp.float32)
    s0v = 1.0 / denomv
    s1v = jnp.exp(jnp.full((16,), m2 - m1, jnp.float32)) / denomv
    s0 = jnp.max(s0v)
    s1 = jnp.max(s1v)

    wid = c * 16 + s
    base = wid * RPW
    idxv[pl.ds(0, 16)] = i0 * D_HID + base + lanes
    idxv[pl.ds(16, 16)] = jnp.where(
        lanes < 8, i0 * D_HID + base + 16 + lanes,
        i1 * D_HID + base + (lanes - 8))
    idxv[pl.ds(32, 16)] = i1 * D_HID + base + 8 + lanes
    oidx[pl.ds(0, 16)] = base + lanes
    oidx[pl.ds(8, 16)] = base + 8 + lanes
    cpg = pltpu.async_copy(wrows_hbm.at[idxv], wbuf, sem0)

    @pl.when((s == 0) & (c == 0))
    def _bias_fetch():
        pltpu.sync_copy(b_hbm.at[i0], bb0)
        pltpu.sync_copy(b_hbm.at[i1], bb1)

    cpg.wait()

    def comb(r, _):
        for u in range(48):
            sl = pl.ds(u * 16, 16)
            wbuf[r, sl] = s0 * wbuf[r, sl] + s1 * wbuf[r + RPW, sl]
        return 0

    lax.fori_loop(0, RPW, comb, 0)
    pltpu.async_copy(wbuf.at[pl.ds(0, RPW)], wc_out.at[oidx], sem1).wait()

    @pl.when((s == 0) & (c == 0))
    def _bias():
        def combb(j, _):
            sl = pl.ds(j * 16, 16)
            bb0[sl] = s0 * bb0[sl] + s1 * bb1[sl]
            return 0

        lax.fori_loop(0, D_HID // 16, combb, 0)
        pltpu.sync_copy(bb0, bc_out)


def _matmul_tc(x_ref, wc_ref, bc_ref, out_ref):
    out_ref[...] = jax.lax.dot_general(
        x_ref[...], wc_ref[...], (((1,), (1,)), ((), ())),
        preferred_element_type=jnp.float32) + bc_ref[...]


def kernel(x, W_experts, b_experts, Wg, bg):
    n_tokens = x.shape[0]
    wgr = (Wg.T.reshape(16, 48, 4, 16).transpose(2, 0, 1, 3)
           .reshape(4, 16, D_IN))
    bgr = bg.reshape(4, 16)
    w_rows = W_experts.reshape(E * D_HID, D_IN)

    sc_fn = pl.kernel(
        _route_combine_sc,
        out_type=(
            jax.ShapeDtypeStruct((D_HID, D_IN), jnp.float32),
            jax.ShapeDtypeStruct((D_HID,), jnp.float32),
        ),
        mesh=plsc.VectorSubcoreMesh(core_axis_name="c", subcore_axis_name="s"),
        compiler_params=pltpu.CompilerParams(needs_layout_passes=False),
        scratch_types=[
            pltpu.VMEM((D_IN,), jnp.float32),
            pltpu.VMEM((16,), jnp.float32),
            pltpu.VMEM((2 * RPW, D_IN), jnp.float32),
            pltpu.VMEM((48,), jnp.int32),
            pltpu.VMEM((RPW,), jnp.int32),
            pltpu.VMEM((D_HID,), jnp.float32),
            pltpu.VMEM((D_HID,), jnp.float32),
            pltpu.SemaphoreType.DMA,
            pltpu.SemaphoreType.DMA,
        ],
    )
    wc, bc = sc_fn(x, wgr, bgr, w_rows, b_experts)

    return pl.pallas_call(
        _matmul_tc,
        grid=(n_tokens // BT,),
        in_specs=[
            pl.BlockSpec((BT, D_IN), lambda i: (i, 0)),
            pl.BlockSpec((D_HID, D_IN), lambda i: (0, 0)),
            pl.BlockSpec((1, D_HID), lambda i: (0, 0)),
        ],
        out_specs=pl.BlockSpec((BT, D_HID), lambda i: (i, 0)),
        out_shape=jax.ShapeDtypeStruct((n_tokens, D_HID), jnp.float32),
    )(x, wc, bc.reshape(1, D_HID))
